# Initial kernel scaffold; baseline (speedup 1.0000x reference)
import jax
import jax.numpy as jnp
from jax.experimental import pallas as pl


def kernel(input_ids, table, W, b):
    raise NotImplementedError("write your pallas kernel here")


# speedup vs baseline: 1.8668x; 1.8668x over previous
import functools

import jax
import jax.numpy as jnp
from jax import lax
from jax.experimental import pallas as pl
from jax.experimental.pallas import tpu as pltpu
from jax.experimental.pallas import tpu_sc as plsc

B = 4096
L = 200
EMB = 32
VOCAB = 1000000

NC = 2
NS = 16
NW = NC * NS
BPW = B // NW

_MV_BLK = 10000


def _mv_body(a_ref, wt_ref, bl_ref, o_ref):
    o_ref[...] = (
        jnp.dot(a_ref[...], wt_ref[...], preferred_element_type=jnp.float32)
        + bl_ref[0, 0]
    )


def _tc_matvec(a, wtile, bl):
    rows = a.shape[0]
    return pl.pallas_call(
        _mv_body,
        grid=(rows // _MV_BLK,),
        in_specs=[
            pl.BlockSpec((_MV_BLK, 4 * EMB), lambda i: (i, 0)),
            pl.BlockSpec((4 * EMB, 4), lambda i: (0, 0)),
            pl.BlockSpec(memory_space=pltpu.SMEM),
        ],
        out_specs=pl.BlockSpec((_MV_BLK, 4), lambda i: (i, 0)),
        out_shape=jax.ShapeDtypeStruct((rows, 4), jnp.float32),
    )(a, wtile, bl)


_mesh = plsc.VectorSubcoreMesh(core_axis_name="c", subcore_axis_name="s")


@functools.partial(
    pl.kernel,
    mesh=_mesh,
    out_type=jax.ShapeDtypeStruct((B,), jnp.float32),
    scratch_types=[
        pltpu.VMEM((L * BPW,), jnp.int32),
        pltpu.VMEM((L * BPW,), jnp.float32),
        pltpu.VMEM((BPW,), jnp.float32),
        pltpu.SemaphoreType.DMA,
    ],
)
def _sc_pool(ids_hbm, tv_hbm, out_hbm, idx_v, vals_v, out_v, sem):
    wid = lax.axis_index("s") * NC + lax.axis_index("c")
    pltpu.sync_copy(ids_hbm.at[wid], idx_v)
    pltpu.async_copy(tv_hbm.at[idx_v], vals_v, sem).wait()

    def body(l, accs):
        return tuple(
            a + vals_v[pl.ds(l * BPW + 16 * j, 16)] for j, a in enumerate(accs)
        )

    zeros = tuple(jnp.zeros((16,), jnp.float32) for _ in range(BPW // 16))
    accs = lax.fori_loop(0, L, body, zeros)
    for j, a in enumerate(accs):
        out_v[pl.ds(16 * j, 16)] = 1.0 / (1.0 + jnp.exp(-a))
    pltpu.sync_copy(out_v, out_hbm.at[pl.ds(wid * BPW, BPW)])


def kernel(input_ids, table, W, b):
    a = table.reshape(VOCAB // 4, 4 * EMB)
    eye4 = jnp.eye(4, dtype=jnp.float32)
    wtile = (W.reshape(-1)[None, :, None] * eye4[:, None, :]).reshape(4 * EMB, 4)
    wtile = wtile * (1.0 / L)
    bl = (b * (1.0 / L)).reshape(1, 1).astype(jnp.float32)

    tv = _tc_matvec(a, wtile, bl).reshape(VOCAB)

    ids_arr = input_ids.reshape(NW, BPW, L).transpose(0, 2, 1).reshape(NW, L * BPW)
    out = _sc_pool(ids_arr, tv)
    return out.reshape(B, 1)


# speedup vs baseline: 2.6763x; 1.4336x over previous
import functools

import jax
import jax.numpy as jnp
from jax import lax
from jax.experimental import pallas as pl
from jax.experimental.pallas import tpu as pltpu
from jax.experimental.pallas import tpu_sc as plsc

B = 4096
L = 200
EMB = 32
VOCAB = 1000000

NC = 2
NS = 16
NW = NC * NS
BPW = B // NW
IPW = BPW * L

_MV_BLK = 32768
_MV_NBLK = -(-VOCAB // _MV_BLK)
_TV_LEN = _MV_NBLK * _MV_BLK


def _mv_body(wr_ref, a_ref, bl_ref, o_ref):
    r = lax.dot_general(
        wr_ref[...], a_ref[...], (((1,), (1,)), ((), ())),
        preferred_element_type=jnp.float32,
    )
    o_ref[...] = (r + bl_ref[0, 0]).reshape(_MV_BLK)


def _tc_matvec(table, wr, bl):
    return pl.pallas_call(
        _mv_body,
        grid=(_MV_NBLK,),
        in_specs=[
            pl.BlockSpec((1, EMB), lambda i: (0, 0)),
            pl.BlockSpec((_MV_BLK, EMB), lambda i: (i, 0)),
            pl.BlockSpec(memory_space=pltpu.SMEM),
        ],
        out_specs=pl.BlockSpec((_MV_BLK,), lambda i: (i,)),
        out_shape=jax.ShapeDtypeStruct((_TV_LEN,), jnp.float32),
    )(wr, table, bl)


_mesh = plsc.VectorSubcoreMesh(core_axis_name="c", subcore_axis_name="s")


@functools.partial(
    pl.kernel,
    mesh=_mesh,
    out_type=jax.ShapeDtypeStruct((B,), jnp.float32),
    compiler_params=pltpu.CompilerParams(needs_layout_passes=False),
    scratch_types=[
        pltpu.VMEM((IPW,), jnp.int32),
        pltpu.VMEM((IPW,), jnp.float32),
        pltpu.VMEM((BPW,), jnp.float32),
        pltpu.SemaphoreType.DMA,
    ],
)
def _sc_pool(ids_hbm, tv_hbm, out_hbm, idx_v, vals_v, out_v, sem):
    wid = lax.axis_index("s") * NC + lax.axis_index("c")
    pltpu.sync_copy(ids_hbm.at[pl.ds(wid * IPW, IPW)], idx_v)
    pltpu.async_copy(tv_hbm.at[idx_v], vals_v, sem).wait()

    lanebase = lax.iota(jnp.int32, 16) * L
    for c in range(BPW // 16):
        def body(l, acc):
            return acc + plsc.load_gather(vals_v, [lanebase + (c * 16 * L + l)])

        acc = lax.fori_loop(0, L, body, jnp.zeros((16,), jnp.float32))
        out_v[pl.ds(c * 16, 16)] = 1.0 / (1.0 + jnp.exp(-acc))
    pltpu.sync_copy(out_v, out_hbm.at[pl.ds(wid * BPW, BPW)])


def kernel(input_ids, table, W, b):
    wr = (W * (1.0 / L)).astype(jnp.float32)
    bl = (b * (1.0 / L)).reshape(1, 1).astype(jnp.float32)

    tv = _tc_matvec(table, wr, bl)
    ids_flat = input_ids.reshape(NW * IPW)
    out = _sc_pool(ids_flat, tv)
    return out.reshape(B, 1)
